# baseline (speedup 1.0000x reference)
import functools

import jax
import jax.numpy as jnp
from jax import lax
from jax.experimental import pallas as pl
from jax.experimental.pallas import tpu as pltpu
from jax.experimental.pallas import tpu_sc as plsc

_NC, _NS = 2, 16
_NW = _NC * _NS
_R = 16
_NXB = 8
_LA = 6


def kernel(x, pos_table):
    batch, seqlen, dim = x.shape
    nrows = batch * seqlen
    x2 = x.reshape(nrows, dim)
    rows_per_w = seqlen // _NW
    nchunk = rows_per_w // _R
    nstage = nchunk * batch
    lanes = 16
    ncol = dim // lanes
    mesh = plsc.VectorSubcoreMesh(
        core_axis_name="c", subcore_axis_name="s", num_cores=_NC, num_subcores=_NS
    )

    @functools.partial(
        pl.kernel,
        out_type=jax.ShapeDtypeStruct((nrows, dim), x.dtype),
        mesh=mesh,
        scratch_types=[
            [pltpu.VMEM((_R, dim), jnp.float32) for _ in range(2)],
            [pltpu.VMEM((_R, dim), jnp.float32) for _ in range(_NXB)],
            [pltpu.SemaphoreType.DMA for _ in range(2)],
            [pltpu.SemaphoreType.DMA for _ in range(_NXB)],
            [pltpu.SemaphoreType.DMA for _ in range(_NXB)],
        ],
    )
    def run(x_hbm, pos_hbm, out_hbm, tbufs, xbufs, sts, sxs, sss):
        wid = lax.axis_index("s") * _NC + lax.axis_index("c")
        sbase = wid * rows_per_w

        def stage_rows(t, b):
            return pl.ds(b * seqlen + sbase + t * _R, _R)

        def load_x(t, b, j):
            pltpu.async_copy(x_hbm.at[stage_rows(t, b)], xbufs[j], sxs[j])

        def wait_load(j):
            pltpu.make_async_copy(x_hbm.at[pl.ds(0, _R)], xbufs[j], sxs[j]).wait()

        def store(t, b, j):
            pltpu.async_copy(xbufs[j], out_hbm.at[stage_rows(t, b)], sss[j])

        def wait_store(j):
            pltpu.make_async_copy(xbufs[j], out_hbm.at[pl.ds(0, _R)], sss[j]).wait()

        def load_table(t, p):
            sb = sbase + lax.rem(t, nchunk) * _R
            pltpu.async_copy(pos_hbm.at[pl.ds(sb, _R)], tbufs[p], sts[p])

        def wait_table(p):
            pltpu.make_async_copy(pos_hbm.at[pl.ds(0, _R)], tbufs[p], sts[p]).wait()

        load_table(0, 0)
        for k in range(_LA):
            load_x(k // batch, k % batch, k % _NXB)

        def body(it, carry):
            for o in range(2 * batch):
                k_off = o
                tpar = o // batch
                b = o % batch
                t = it * 2 + tpar
                if b == 0:
                    wait_table(tpar)
                    load_table(t + 1, 1 - tpar)
                j = o % _NXB
                wait_load(j)
                ka_off = o + _LA
                ja = ka_off % _NXB
                ta = it * 2 + ka_off // batch
                ba = ka_off % batch

                @pl.when(ta * batch + ba < nstage)
                def _():
                    @pl.when(it * 2 * batch + o >= _NXB - _LA)
                    def _():
                        wait_store(ja)
                    load_x(ta, ba, ja)

                tbuf = tbufs[tpar]
                xbuf = xbufs[j]

                def addrow(r, c):
                    for cj in range(ncol):
                        sl = pl.ds(cj * lanes, lanes)
                        plsc.addupdate(xbuf.at[r, sl], tbuf[r, sl])
                    return c

                lax.fori_loop(0, _R, addrow, 0)
                store(t, b, j)
            return carry

        lax.fori_loop(0, nchunk // 2, body, 0)

        for j in range(_NXB):
            wait_store(j)
        wait_table(0)

    return run(x2, pos_table).reshape(x.shape)


# speedup vs baseline: 1.0080x; 1.0080x over previous
import functools

import jax
import jax.numpy as jnp
from jax import lax
from jax.experimental import pallas as pl
from jax.experimental.pallas import tpu as pltpu
from jax.experimental.pallas import tpu_sc as plsc

_NC, _NS = 2, 16
_NW = _NC * _NS
_R = 16
_NXB = 8
_LA = 4


def kernel(x, pos_table):
    batch, seqlen, dim = x.shape
    nrows = batch * seqlen
    x2 = x.reshape(nrows, dim)
    rows_per_w = seqlen // _NW
    nchunk = rows_per_w // _R
    nstage = nchunk * batch
    lanes = 16
    ncol = dim // lanes
    mesh = plsc.VectorSubcoreMesh(
        core_axis_name="c", subcore_axis_name="s", num_cores=_NC, num_subcores=_NS
    )

    @functools.partial(
        pl.kernel,
        out_type=jax.ShapeDtypeStruct((nrows, dim), x.dtype),
        mesh=mesh,
        scratch_types=[
            [pltpu.VMEM((_R, dim), jnp.float32) for _ in range(2)],
            [pltpu.VMEM((_R, dim), jnp.float32) for _ in range(_NXB)],
            [pltpu.SemaphoreType.DMA for _ in range(2)],
            [pltpu.SemaphoreType.DMA for _ in range(_NXB)],
            [pltpu.SemaphoreType.DMA for _ in range(_NXB)],
        ],
    )
    def run(x_hbm, pos_hbm, out_hbm, tbufs, xbufs, sts, sxs, sss):
        wid = lax.axis_index("s") * _NC + lax.axis_index("c")
        sbase = wid * rows_per_w

        def stage_rows(t, b):
            return pl.ds(b * seqlen + sbase + t * _R, _R)

        def load_x(t, b, j):
            pltpu.async_copy(x_hbm.at[stage_rows(t, b)], xbufs[j], sxs[j])

        def wait_load(j):
            pltpu.make_async_copy(x_hbm.at[pl.ds(0, _R)], xbufs[j], sxs[j]).wait()

        def store(t, b, j):
            pltpu.async_copy(xbufs[j], out_hbm.at[stage_rows(t, b)], sss[j])

        def wait_store(j):
            pltpu.make_async_copy(xbufs[j], out_hbm.at[pl.ds(0, _R)], sss[j]).wait()

        def load_table(t, p):
            sb = sbase + lax.rem(t, nchunk) * _R
            pltpu.async_copy(pos_hbm.at[pl.ds(sb, _R)], tbufs[p], sts[p])

        def wait_table(p):
            pltpu.make_async_copy(pos_hbm.at[pl.ds(0, _R)], tbufs[p], sts[p]).wait()

        load_table(0, 0)
        for k in range(_LA):
            load_x(k // batch, k % batch, k % _NXB)

        def body(it, carry):
            for o in range(2 * batch):
                k_off = o
                tpar = o // batch
                b = o % batch
                t = it * 2 + tpar
                if b == 0:
                    wait_table(tpar)
                    load_table(t + 1, 1 - tpar)
                j = o % _NXB
                wait_load(j)
                ka_off = o + _LA
                ja = ka_off % _NXB
                ta = it * 2 + ka_off // batch
                ba = ka_off % batch

                @pl.when(ta * batch + ba < nstage)
                def _():
                    @pl.when(it * 2 * batch + o >= _NXB - _LA)
                    def _():
                        wait_store(ja)
                    load_x(ta, ba, ja)

                tbuf = tbufs[tpar]
                xbuf = xbufs[j]

                def addrow(r, c):
                    for cj in range(ncol):
                        sl = pl.ds(cj * lanes, lanes)
                        plsc.addupdate(xbuf.at[r, sl], tbuf[r, sl])
                    return c

                lax.fori_loop(0, _R, addrow, 0)
                store(t, b, j)
            return carry

        lax.fori_loop(0, nchunk // 2, body, 0)

        for j in range(_NXB):
            wait_store(j)
        wait_table(0)

    return run(x2, pos_table).reshape(x.shape)
